# baseline (speedup 1.0000x reference)
import functools

import jax
import jax.numpy as jnp
from jax import lax
from jax.experimental import pallas as pl
from jax.experimental.pallas import tpu as pltpu
from jax.experimental.pallas import tpu_sc as plsc

_T = 2048
_H = 1024
_E = 64
_CAP = 128
_S = _E * _CAP


def _iround(x):
    return jnp.floor(x + 0.5)


def _router_body(flat_ref, gate_ref, tbl_ref, occ_ref, inv_ref, cnt_ref):
    flat = flat_ref[...]
    gate = gate_ref[...]
    logits = lax.dot_general(flat, gate, (((1,), (1,)), ((), ())),
                             preferred_element_type=jnp.float32)
    m = jnp.max(logits, axis=1, keepdims=True)
    iota_e = lax.broadcasted_iota(jnp.int32, (_T, _E), 1).astype(jnp.float32)
    eidf = jnp.min(jnp.where(logits >= m, iota_e, float(_E)), axis=1,
                   keepdims=True)
    one_hot = (iota_e == eidf).astype(jnp.float32)

    r = lax.broadcasted_iota(jnp.int32, (_CAP, _CAP), 0)
    c = lax.broadcasted_iota(jnp.int32, (_CAP, _CAP), 1)
    ltri = (c <= r).astype(jnp.float32)
    nchunk = _T // _CAP
    r_parts, s_parts = [], []
    for k in range(nchunk):
        chunk = one_hot[k * _CAP:(k + 1) * _CAP, :]
        rk = jnp.dot(ltri, chunk, preferred_element_type=jnp.float32)
        r_parts.append(rk)
        s_parts.append(rk[_CAP - 1:_CAP, :])
    r_in = jnp.concatenate(r_parts, axis=0)
    s_tot = jnp.concatenate(s_parts, axis=0)
    kr = lax.broadcasted_iota(jnp.int32, (nchunk, nchunk), 0)
    kc = lax.broadcasted_iota(jnp.int32, (nchunk, nchunk), 1)
    stri = (kc < kr).astype(jnp.float32)
    pexc = jnp.dot(stri, s_tot, preferred_element_type=jnp.float32)
    tch = lax.broadcasted_iota(jnp.int32, (_T, nchunk), 0) // _CAP
    kk = lax.broadcasted_iota(jnp.int32, (_T, nchunk), 1)
    u = (tch == kk).astype(jnp.float32)
    r_all = r_in + jnp.dot(u, pexc, preferred_element_type=jnp.float32)
    rank = _iround(jnp.sum(r_all * one_hot, axis=1, keepdims=True) - 1.0)

    tvec = lax.broadcasted_iota(jnp.int32, (_T, 1), 0).astype(jnp.float32)
    iota_c = lax.broadcasted_iota(jnp.int32, (_T, _CAP), 1).astype(jnp.float32)
    hit = (iota_c == rank).astype(jnp.float32)
    tbl_f = lax.dot_general(one_hot * tvec, hit, (((0,), (0,)), ((), ())),
                            preferred_element_type=jnp.float32)
    occ_ec = _iround(lax.dot_general(one_hot, hit, (((0,), (0,)), ((), ())),
                                     preferred_element_type=jnp.float32))
    e_iota = lax.broadcasted_iota(jnp.int32, (_E, _CAP), 0)
    c_iota = lax.broadcasted_iota(jnp.int32, (_E, _CAP), 1)
    fill = ((e_iota * _CAP + c_iota) % _T).astype(jnp.float32)
    tbl_ref[...] = (_iround(tbl_f) * occ_ec
                    + fill * (1.0 - occ_ec)).astype(jnp.int32)

    cnt = jnp.sum(one_hot, axis=0, keepdims=True)
    cnt_ref[...] = cnt.astype(jnp.int32)

    cnt_col = lax.dot_general(one_hot, jnp.ones((_T, 1), jnp.float32),
                              (((0,), (0,)), ((), ())),
                              preferred_element_type=jnp.float32)
    s_iota = lax.broadcasted_iota(jnp.int32, (_S, 1), 0)
    es = (s_iota // _CAP).astype(jnp.float32)
    cs = (s_iota % _CAP).astype(jnp.float32)
    e_cols = lax.broadcasted_iota(jnp.int32, (_S, _E), 1).astype(jnp.float32)
    u_s = (e_cols == es).astype(jnp.float32)
    cnt_per_s = jnp.dot(u_s, cnt_col, preferred_element_type=jnp.float32)
    occ_ref[...] = (cs < cnt_per_s - 0.5).astype(jnp.float32)

    iota_e1 = lax.broadcasted_iota(jnp.int32, (1, _E), 1).astype(jnp.float32)
    cmin = jnp.min(cnt)
    estar = jnp.min(jnp.where(cnt == cmin, iota_e1, float(_E)))
    inv_f = jnp.where(rank < float(_CAP), eidf * _CAP + rank,
                      estar * _CAP + (_CAP - 1))
    inv_ref[...] = _iround(inv_f).astype(jnp.int32)


_ROUTER_KW = dict(
    out_shape=(
        jax.ShapeDtypeStruct((_E, _CAP), jnp.int32),
        jax.ShapeDtypeStruct((_S, 1), jnp.float32),
        jax.ShapeDtypeStruct((_T, 1), jnp.int32),
        jax.ShapeDtypeStruct((1, _E), jnp.int32),
    ),
)


def _mlp_body(occ_ref, xg_ref, wg_ref, wu_ref, wd_ref, y_ref):
    x = xg_ref[...]
    wg = wg_ref[0]
    wu = wu_ref[0]
    wd = wd_ref[0]
    g = lax.dot_general(x, wg, (((1,), (1,)), ((), ())),
                        preferred_element_type=jnp.float32)
    uu = lax.dot_general(x, wu, (((1,), (1,)), ((), ())),
                         preferred_element_type=jnp.float32)
    h = g * (1.0 / (1.0 + jnp.exp(-g))) * uu
    y = lax.dot_general(h, wd, (((1,), (1,)), ((), ())),
                        preferred_element_type=jnp.float32)
    y_ref[...] = jnp.where(occ_ref[...] > 0.5, y, 0.0)


def _mlp_body_alias(occ_ref, xg_ref, wg_ref, wu_ref, wd_ref, yprev_ref, y_ref):
    _mlp_body(occ_ref, xg_ref, wg_ref, wu_ref, wd_ref, y_ref)


def _mlp_kw(I, H, e_lo, n_exp, alias):
    in_specs = [
        pl.BlockSpec((_CAP, 1), lambda e: (e + e_lo, 0)),
        pl.BlockSpec((_CAP, H), lambda e: (e, 0)),
        pl.BlockSpec((1, I, H), lambda e: (e + e_lo, 0, 0)),
        pl.BlockSpec((1, I, H), lambda e: (e + e_lo, 0, 0)),
        pl.BlockSpec((1, H, I), lambda e: (e + e_lo, 0, 0)),
    ]
    kw = dict(
        grid=(n_exp,),
        out_specs=pl.BlockSpec((_CAP, H), lambda e: (e + e_lo, 0)),
        out_shape=jax.ShapeDtypeStruct((_S, H), jnp.float32),
    )
    if alias:
        in_specs.append(pl.BlockSpec(memory_space=pltpu.MemorySpace.HBM))
        kw["input_output_aliases"] = {5: 0}
    kw["in_specs"] = in_specs
    return kw


def _make_sc_gather(n_rows, n_cols, n_out):
    info = plsc.get_sparse_core_info()
    n_workers = info.num_cores * info.num_subcores
    per_w = n_out // n_workers
    chunk = min(per_w, 64)
    nchunks = per_w // chunk
    mesh = plsc.VectorSubcoreMesh(core_axis_name="c", subcore_axis_name="s")

    @functools.partial(
        pl.kernel, mesh=mesh,
        out_type=jax.ShapeDtypeStruct((n_out, n_cols), jnp.float32),
        scratch_types=[
            pltpu.VMEM((chunk,), jnp.int32),
            pltpu.VMEM((chunk, n_cols), jnp.float32),
            pltpu.SemaphoreType.DMA,
        ],
    )
    def gather(table_hbm, idx_hbm, out_hbm, idx_v, rows_v, sem):
        wid = lax.axis_index("s") * info.num_cores + lax.axis_index("c")
        base = wid * per_w
        for ci in range(nchunks):
            off = base + ci * chunk
            pltpu.sync_copy(idx_hbm.at[pl.ds(off, chunk)], idx_v)
            pltpu.async_copy(table_hbm.at[idx_v], rows_v, sem).wait()
            pltpu.sync_copy(rows_v, out_hbm.at[pl.ds(off, chunk)])

    return gather


def _make_sc_gather_dyn():
    info = plsc.get_sparse_core_info()
    n_workers = info.num_cores * info.num_subcores
    epw = _E // n_workers
    chunk = 32
    max_chunks = _CAP // chunk
    slots_w = epw * _CAP
    mesh = plsc.VectorSubcoreMesh(core_axis_name="c", subcore_axis_name="s")

    @functools.partial(
        pl.kernel, mesh=mesh,
        out_type=jax.ShapeDtypeStruct((_S, _H), jnp.float32),
        scratch_types=[
            pltpu.VMEM((slots_w,), jnp.int32),
            pltpu.VMEM((chunk, _H), jnp.float32),
            pltpu.VMEM((_E + 16,), jnp.int32),
            pltpu.SemaphoreType.DMA,
        ],
    )
    def gather(table_hbm, idx_hbm, cnt_hbm, out_hbm, idx_v, rows_v, cnt_v, sem):
        wid = lax.axis_index("s") * info.num_cores + lax.axis_index("c")
        base = wid * slots_w
        pltpu.sync_copy(idx_hbm.at[pl.ds(base, slots_w)], idx_v)
        pltpu.sync_copy(cnt_hbm, cnt_v.at[pl.ds(0, _E)])
        for j in range(epw):
            e = wid * epw + j
            cnt = cnt_v[pl.ds(e, 16)][0]
            for ci in range(max_chunks):
                @pl.when(ci * chunk < cnt)
                def _do_chunk(ci=ci, j=j, e=e):
                    loc = j * _CAP + ci * chunk
                    pltpu.async_copy(
                        table_hbm.at[idx_v.at[pl.ds(loc, chunk)]],
                        rows_v, sem).wait()
                    pltpu.sync_copy(rows_v, out_hbm.at[pl.ds(base + loc, chunk)])

    return gather


def kernel(hidden_states, gate_w, Wg, Wu, Wd):
    b, t, h = hidden_states.shape
    i_dim = Wg.shape[1]
    flat = hidden_states.reshape(t, h)
    tbl, occ, inv, cnt = pl.pallas_call(
        _router_body, **_ROUTER_KW)(flat, gate_w)
    xg = _make_sc_gather_dyn()(flat, tbl.reshape(_S), cnt.reshape(_E))
    y = pl.pallas_call(_mlp_body, **_mlp_kw(i_dim, h, 0, _E, False))(
        occ, xg, Wg, Wu, Wd)
    out = _make_sc_gather(_S, _H, _T)(y, inv.reshape(_T))
    return out.reshape(b, t, h)


# speedup vs baseline: 1.0061x; 1.0061x over previous
import functools

import jax
import jax.numpy as jnp
from jax import lax
from jax.experimental import pallas as pl
from jax.experimental.pallas import tpu as pltpu
from jax.experimental.pallas import tpu_sc as plsc

_T = 2048
_H = 1024
_E = 64
_CAP = 128
_S = _E * _CAP


def _iround(x):
    return jnp.floor(x + 0.5)


def _router_body(flat_ref, gate_ref, tbl_ref, occ_ref, inv_ref, cnt_ref):
    flat = flat_ref[...]
    gate = gate_ref[...]
    logits = lax.dot_general(flat, gate, (((1,), (1,)), ((), ())),
                             preferred_element_type=jnp.float32)
    m = jnp.max(logits, axis=1, keepdims=True)
    iota_e = lax.broadcasted_iota(jnp.int32, (_T, _E), 1).astype(jnp.float32)
    eidf = jnp.min(jnp.where(logits >= m, iota_e, float(_E)), axis=1,
                   keepdims=True)
    one_hot = (iota_e == eidf).astype(jnp.float32)

    r = lax.broadcasted_iota(jnp.int32, (_CAP, _CAP), 0)
    c = lax.broadcasted_iota(jnp.int32, (_CAP, _CAP), 1)
    ltri = (c <= r).astype(jnp.float32)
    nchunk = _T // _CAP
    r_parts, s_parts = [], []
    for k in range(nchunk):
        chunk = one_hot[k * _CAP:(k + 1) * _CAP, :]
        rk = jnp.dot(ltri, chunk, preferred_element_type=jnp.float32)
        r_parts.append(rk)
        s_parts.append(rk[_CAP - 1:_CAP, :])
    r_in = jnp.concatenate(r_parts, axis=0)
    s_tot = jnp.concatenate(s_parts, axis=0)
    kr = lax.broadcasted_iota(jnp.int32, (nchunk, nchunk), 0)
    kc = lax.broadcasted_iota(jnp.int32, (nchunk, nchunk), 1)
    stri = (kc < kr).astype(jnp.float32)
    pexc = jnp.dot(stri, s_tot, preferred_element_type=jnp.float32)
    tch = lax.broadcasted_iota(jnp.int32, (_T, nchunk), 0) // _CAP
    kk = lax.broadcasted_iota(jnp.int32, (_T, nchunk), 1)
    u = (tch == kk).astype(jnp.float32)
    r_all = r_in + jnp.dot(u, pexc, preferred_element_type=jnp.float32)
    rank = _iround(jnp.sum(r_all * one_hot, axis=1, keepdims=True) - 1.0)

    tvec = lax.broadcasted_iota(jnp.int32, (_T, 1), 0).astype(jnp.float32)
    iota_c = lax.broadcasted_iota(jnp.int32, (_T, _CAP), 1).astype(jnp.float32)
    hit = (iota_c == rank).astype(jnp.float32)
    tbl_f = lax.dot_general(one_hot * tvec, hit, (((0,), (0,)), ((), ())),
                            preferred_element_type=jnp.float32)
    occ_ec = _iround(lax.dot_general(one_hot, hit, (((0,), (0,)), ((), ())),
                                     preferred_element_type=jnp.float32))
    e_iota = lax.broadcasted_iota(jnp.int32, (_E, _CAP), 0)
    c_iota = lax.broadcasted_iota(jnp.int32, (_E, _CAP), 1)
    fill = ((e_iota * _CAP + c_iota) % _T).astype(jnp.float32)
    tbl_ref[...] = (_iround(tbl_f) * occ_ec
                    + fill * (1.0 - occ_ec)).astype(jnp.int32)

    cnt = jnp.sum(one_hot, axis=0, keepdims=True)
    cnt_ref[...] = cnt.astype(jnp.int32)

    cnt_col = lax.dot_general(one_hot, jnp.ones((_T, 1), jnp.float32),
                              (((0,), (0,)), ((), ())),
                              preferred_element_type=jnp.float32)
    s_iota = lax.broadcasted_iota(jnp.int32, (_S, 1), 0)
    es = (s_iota // _CAP).astype(jnp.float32)
    cs = (s_iota % _CAP).astype(jnp.float32)
    e_cols = lax.broadcasted_iota(jnp.int32, (_S, _E), 1).astype(jnp.float32)
    u_s = (e_cols == es).astype(jnp.float32)
    cnt_per_s = jnp.dot(u_s, cnt_col, preferred_element_type=jnp.float32)
    occ_ref[...] = (cs < cnt_per_s - 0.5).astype(jnp.float32)

    iota_e1 = lax.broadcasted_iota(jnp.int32, (1, _E), 1).astype(jnp.float32)
    cmin = jnp.min(cnt)
    estar = jnp.min(jnp.where(cnt == cmin, iota_e1, float(_E)))
    inv_f = jnp.where(rank < float(_CAP), eidf * _CAP + rank,
                      estar * _CAP + (_CAP - 1))
    inv_ref[...] = _iround(inv_f).astype(jnp.int32)


_ROUTER_KW = dict(
    out_shape=(
        jax.ShapeDtypeStruct((_E, _CAP), jnp.int32),
        jax.ShapeDtypeStruct((_S, 1), jnp.float32),
        jax.ShapeDtypeStruct((_T, 1), jnp.int32),
        jax.ShapeDtypeStruct((1, _E), jnp.int32),
    ),
)


def _mlp_body(occ_ref, xg_ref, wg_ref, wu_ref, wd_ref, y_ref):
    x = xg_ref[...]
    wg = wg_ref[0]
    wu = wu_ref[0]
    wd = wd_ref[0]
    g = lax.dot_general(x, wg, (((1,), (1,)), ((), ())),
                        preferred_element_type=jnp.float32)
    uu = lax.dot_general(x, wu, (((1,), (1,)), ((), ())),
                         preferred_element_type=jnp.float32)
    h = g * (1.0 / (1.0 + jnp.exp(-g))) * uu
    y = lax.dot_general(h, wd, (((1,), (1,)), ((), ())),
                        preferred_element_type=jnp.float32)
    y_ref[...] = jnp.where(occ_ref[...] > 0.5, y, 0.0)


def _mlp_kw(I, H):
    return dict(
        grid=(_E,),
        in_specs=[
            pl.BlockSpec((_CAP, 1), lambda e: (e, 0)),
            pl.BlockSpec((_CAP, H), lambda e: (e, 0)),
            pl.BlockSpec((1, I, H), lambda e: (e, 0, 0)),
            pl.BlockSpec((1, I, H), lambda e: (e, 0, 0)),
            pl.BlockSpec((1, H, I), lambda e: (e, 0, 0)),
        ],
        out_specs=pl.BlockSpec((_CAP, H), lambda e: (e, 0)),
        out_shape=jax.ShapeDtypeStruct((_S, H), jnp.float32),
    )


def _make_sc_gather(n_rows, n_cols, n_out):
    info = plsc.get_sparse_core_info()
    n_workers = info.num_cores * info.num_subcores
    per_w = n_out // n_workers
    chunk = min(per_w, 64)
    nchunks = per_w // chunk
    mesh = plsc.VectorSubcoreMesh(core_axis_name="c", subcore_axis_name="s")

    @functools.partial(
        pl.kernel, mesh=mesh,
        out_type=jax.ShapeDtypeStruct((n_out, n_cols), jnp.float32),
        scratch_types=[
            pltpu.VMEM((chunk,), jnp.int32),
            pltpu.VMEM((chunk, n_cols), jnp.float32),
            pltpu.SemaphoreType.DMA,
        ],
    )
    def gather(table_hbm, idx_hbm, out_hbm, idx_v, rows_v, sem):
        wid = lax.axis_index("s") * info.num_cores + lax.axis_index("c")
        base = wid * per_w
        for ci in range(nchunks):
            off = base + ci * chunk
            pltpu.sync_copy(idx_hbm.at[pl.ds(off, chunk)], idx_v)
            pltpu.async_copy(table_hbm.at[idx_v], rows_v, sem).wait()
            pltpu.sync_copy(rows_v, out_hbm.at[pl.ds(off, chunk)])

    return gather


def _make_sc_gather_dyn():
    info = plsc.get_sparse_core_info()
    n_workers = info.num_cores * info.num_subcores
    epw = _E // n_workers
    chunk = 32
    max_chunks = _CAP // chunk
    slots_w = epw * _CAP
    mesh = plsc.VectorSubcoreMesh(core_axis_name="c", subcore_axis_name="s")

    @functools.partial(
        pl.kernel, mesh=mesh,
        out_type=jax.ShapeDtypeStruct((_S, _H), jnp.float32),
        scratch_types=[
            pltpu.VMEM((slots_w,), jnp.int32),
            pltpu.VMEM((chunk, _H), jnp.float32),
            pltpu.VMEM((_E + 16,), jnp.int32),
            pltpu.SemaphoreType.DMA,
        ],
    )
    def gather(table_hbm, idx_hbm, cnt_hbm, out_hbm, idx_v, rows_v, cnt_v, sem):
        wid = lax.axis_index("s") * info.num_cores + lax.axis_index("c")
        base = wid * slots_w
        pltpu.sync_copy(idx_hbm.at[pl.ds(base, slots_w)], idx_v)
        pltpu.sync_copy(cnt_hbm, cnt_v.at[pl.ds(0, _E)])
        for j in range(epw):
            e = wid * epw + j
            cnt = cnt_v[pl.ds(e, 16)][0]
            for ci in range(max_chunks):
                @pl.when(ci * chunk < cnt)
                def _do_chunk(ci=ci, j=j, e=e):
                    loc = j * _CAP + ci * chunk
                    pltpu.async_copy(
                        table_hbm.at[idx_v.at[pl.ds(loc, chunk)]],
                        rows_v, sem).wait()
                    pltpu.sync_copy(rows_v, out_hbm.at[pl.ds(base + loc, chunk)])

    return gather


def kernel(hidden_states, gate_w, Wg, Wu, Wd):
    b, t, h = hidden_states.shape
    i_dim = Wg.shape[1]
    flat = hidden_states.reshape(t, h)
    tbl, occ, inv, cnt = pl.pallas_call(
        _router_body, **_ROUTER_KW)(flat, gate_w)
    xg = _make_sc_gather_dyn()(flat, tbl.reshape(_S), cnt.reshape(_E))
    y = pl.pallas_call(_mlp_body, **_mlp_kw(i_dim, h))(occ, xg, Wg, Wu, Wd)
    out = _make_sc_gather(_S, _H, _T)(y, inv.reshape(_T))
    return out.reshape(b, t, h)
